# Initial kernel scaffold; baseline (speedup 1.0000x reference)
import jax
import jax.numpy as jnp
from jax.experimental import pallas as pl


def kernel(x, w1, b1, w2, b2, wl1, bl1, wl2, bl2, wl3, bl3, p1, s2, p2):
    raise NotImplementedError("write your pallas kernel here")


# speedup vs baseline: 7.6430x; 7.6430x over previous
import numpy as np
import jax
import jax.numpy as jnp
from jax.experimental import pallas as pl
from jax.experimental.pallas import tpu as pltpu

BI = 64
_KROW = 32


def _build_t1():
    t1 = np.zeros((5 * _KROW, 24, 25), np.float32)
    for r in range(5):
        for j in range(24):
            for dk in range(5):
                t1[r * _KROW + j + dk, j, r * 5 + dk] = 1.0
    return t1


def _build_t2():
    t2 = np.zeros((60, 8, 25), np.float32)
    for r in range(5):
        for j2 in range(8):
            for dk in range(5):
                t2[r * 12 + j2 + dk, j2, r * 5 + dk] = 1.0
    return t2


_T1 = _build_t1()
_T2 = _build_t2()


def _digit5_kernel(x_ref, w1b_ref, b1t_ref, w2b_ref, b2t_ref,
                   wl1_ref, bl1_ref, wl2_ref, bl2_ref, wl3_ref, bl3_ref,
                   out_ref):
    f32 = jnp.float32
    x = x_ref[...]

    p1 = jnp.concatenate([x[:, r:r + 24, :] for r in range(5)], axis=2)
    p1 = p1.reshape(BI * 24, 5 * _KROW)
    h1 = jnp.dot(p1, w1b_ref[...], preferred_element_type=f32) + b1t_ref[...]
    h1 = h1.reshape(BI, 12, 2, 1536)
    h1 = jnp.max(h1, axis=2)
    h1 = h1.reshape(BI, 12, 12, 2, 64)
    h1 = jnp.max(h1, axis=3)
    pooled1 = jnp.maximum(h1, 0.0)

    p2 = jnp.concatenate([pooled1[:, r:r + 8] for r in range(5)], axis=2)
    p2 = p2.reshape(BI * 8, 3840)
    h2 = jnp.dot(p2, w2b_ref[...], preferred_element_type=f32) + b2t_ref[...]
    h2 = h2.reshape(BI, 4, 2, 400)
    h2 = jnp.max(h2, axis=2)
    h2 = h2.reshape(BI, 4, 4, 2, 50)
    h2 = jnp.max(h2, axis=3)
    feats = jnp.maximum(h2, 0.0).reshape(BI, 800)

    h = jnp.dot(feats, wl1_ref[...], preferred_element_type=f32) + bl1_ref[...]
    h = jnp.maximum(h, 0.0)
    h = jnp.dot(h, wl2_ref[...], preferred_element_type=f32) + bl2_ref[...]
    h = jnp.maximum(h, 0.0)
    z = jnp.dot(h, wl3_ref[...], preferred_element_type=f32) + bl3_ref[...]
    m = jnp.max(z, axis=-1, keepdims=True)
    lse = jnp.log(jnp.sum(jnp.exp(z - m), axis=-1, keepdims=True)) + m
    out_ref[...] = z - lse


def kernel(x, w1, b1, w2, b2, wl1, bl1, wl2, bl2, wl3, bl3, p1, s2, p2):
    B = x.shape[0]
    xr = x.reshape(B, 28, 28).astype(jnp.float32)
    xp = jnp.pad(xr, ((0, 0), (0, 0), (0, _KROW - 28)))

    w1eff = jnp.sum(w1, axis=1)
    w1band = jnp.einsum("kjt,to->kjo", _T1, w1eff).reshape(5 * _KROW, 1536)
    w2band = jnp.einsum("ajt,tco->acjo", _T2, w2).reshape(3840, 400)
    b1t = jnp.tile(b1, (1, 24))
    b2t = jnp.tile(b2, (1, 8))

    const = lambda *ndim: (lambda b: tuple(0 for _ in range(len(ndim))))
    in_specs = [
        pl.BlockSpec((BI, 28, _KROW), lambda b: (b, 0, 0)),
        pl.BlockSpec((5 * _KROW, 1536), lambda b: (0, 0)),
        pl.BlockSpec((1, 1536), lambda b: (0, 0)),
        pl.BlockSpec((3840, 400), lambda b: (0, 0)),
        pl.BlockSpec((1, 400), lambda b: (0, 0)),
        pl.BlockSpec((800, 100), lambda b: (0, 0)),
        pl.BlockSpec((1, 100), lambda b: (0, 0)),
        pl.BlockSpec((100, 100), lambda b: (0, 0)),
        pl.BlockSpec((1, 100), lambda b: (0, 0)),
        pl.BlockSpec((100, 10), lambda b: (0, 0)),
        pl.BlockSpec((1, 10), lambda b: (0, 0)),
    ]
    return pl.pallas_call(
        _digit5_kernel,
        out_shape=jax.ShapeDtypeStruct((B, 10), jnp.float32),
        grid=(B // BI,),
        in_specs=in_specs,
        out_specs=pl.BlockSpec((BI, 10), lambda b: (b, 0)),
        compiler_params=pltpu.CompilerParams(
            dimension_semantics=("parallel",),
            vmem_limit_bytes=60 * 1024 * 1024,
        ),
    )(xp, w1band, b1t, w2band, b2t, wl1, bl1, wl2, bl2, wl3, bl3)
